# baseline (speedup 1.0000x reference)
import functools

import jax
import jax.numpy as jnp
from jax import lax
from jax.experimental import pallas as pl
from jax.experimental.pallas import tpu as pltpu
from jax.experimental.pallas import tpu_sc as plsc

N = 524288
D = 128
C = 64

NC = 2
NS = 16
W = NC * NS
ROWS_PER_W = N // W
CH = 128
CHW = CH * D
CHUNKS = ROWS_PER_W // CH
NBUF = 4


def _sc_body(x_hbm, cls_hbm, o_hbm,
             b0, b1, b2, b3, cls_v,
             si0, si1, si2, si3, so0, so1, so2, so3):
    bufs = (b0, b1, b2, b3)
    sins = (si0, si1, si2, si3)
    souts = (so0, so1, so2, so3)

    cid = lax.axis_index("c")
    sid = lax.axis_index("s")
    wid = sid * NC + cid
    base = wid * (ROWS_PER_W * D)

    pltpu.sync_copy(cls_hbm, cls_v)
    lo = cls_v[pl.ds(0, 16)][0]
    hi = cls_v[pl.ds(C - 16, 16)][15]

    def in_slice(g):
        return x_hbm.at[pl.ds(base + g * CHW, CHW)]

    def out_slice(g):
        return o_hbm.at[pl.ds(base + g * CHW, CHW)]

    lane_off = lax.iota(jnp.int32, 16) * D + 5

    def scan_fix(buf):
        @pl.loop(0, CH // 16)
        def _(sb):
            goff = sb * (16 * D)
            vals = plsc.load_gather(buf, [lane_off + goff])
            t = vals.astype(jnp.int32).astype(jnp.float32)
            ok = (vals == t) & (vals >= lo) & (vals <= hi)

            @pl.when(jnp.logical_not(jnp.all(ok)))
            def _():
                @pl.loop(0, 16)
                def _(k):
                    roff = goff + k * D
                    v = buf[pl.ds(roff, 16)][5]
                    vt = v.astype(jnp.int32).astype(jnp.float32)
                    good = (v == vt) & (v >= lo) & (v <= hi)

                    @pl.when(jnp.logical_not(good))
                    def _():
                        for j in range(D // 16):
                            buf[pl.ds(roff + 16 * j, 16)] = jnp.zeros(
                                (16,), jnp.float32)

    for p in range(NBUF - 1):
        pltpu.async_copy(in_slice(p), bufs[p], sins[p])

    @pl.loop(0, CHUNKS, step=NBUF)
    def _(g0):
        for p in range(NBUF):
            g = g0 + p
            buf, si, so = bufs[p], sins[p], souts[p]

            @pl.when(g + NBUF - 1 < CHUNKS)
            def _():
                @pl.when(g >= 1)
                def _():
                    pg = g - 1
                    pltpu.make_async_copy(
                        bufs[(p + NBUF - 1) % NBUF],
                        out_slice(pg),
                        souts[(p + NBUF - 1) % NBUF]).wait()

                pltpu.async_copy(in_slice(g + NBUF - 1),
                                 bufs[(p + NBUF - 1) % NBUF],
                                 sins[(p + NBUF - 1) % NBUF])

            pltpu.make_async_copy(in_slice(g), buf, si).wait()
            scan_fix(buf)
            pltpu.async_copy(buf, out_slice(g), so)

    for t in range(NBUF):
        g = CHUNKS - NBUF + t
        pltpu.make_async_copy(bufs[g % NBUF], out_slice(g),
                              souts[g % NBUF]).wait()


def kernel(x, classes):
    x1d = x.reshape(N * D)
    mesh = plsc.VectorSubcoreMesh(core_axis_name="c", subcore_axis_name="s")
    sc = pl.kernel(
        _sc_body,
        out_type=jax.ShapeDtypeStruct((N * D,), jnp.float32),
        mesh=mesh,
        compiler_params=pltpu.CompilerParams(needs_layout_passes=False),
        scratch_types=[
            pltpu.VMEM((CHW,), jnp.float32),
            pltpu.VMEM((CHW,), jnp.float32),
            pltpu.VMEM((CHW,), jnp.float32),
            pltpu.VMEM((CHW,), jnp.float32),
            pltpu.VMEM((C,), jnp.float32),
            pltpu.SemaphoreType.DMA,
            pltpu.SemaphoreType.DMA,
            pltpu.SemaphoreType.DMA,
            pltpu.SemaphoreType.DMA,
            pltpu.SemaphoreType.DMA,
            pltpu.SemaphoreType.DMA,
            pltpu.SemaphoreType.DMA,
            pltpu.SemaphoreType.DMA,
        ],
    )
    return sc(x1d, classes).reshape(N, D)


# speedup vs baseline: 1.0054x; 1.0054x over previous
import functools

import jax
import jax.numpy as jnp
from jax import lax
from jax.experimental import pallas as pl
from jax.experimental.pallas import tpu as pltpu
from jax.experimental.pallas import tpu_sc as plsc

N = 524288
D = 128
C = 64

NC = 2
NS = 16
W = NC * NS
ROWS_PER_W = N // W
CH = 128
CHW = CH * D
CHUNKS = ROWS_PER_W // CH
NBUF = 4
PD = 2


def _sc_body(x_hbm, cls_hbm, o_hbm,
             b0, b1, b2, b3, cls_v,
             si0, si1, si2, si3, so0, so1, so2, so3):
    bufs = (b0, b1, b2, b3)
    sins = (si0, si1, si2, si3)
    souts = (so0, so1, so2, so3)

    cid = lax.axis_index("c")
    sid = lax.axis_index("s")
    wid = sid * NC + cid
    base = wid * (ROWS_PER_W * D)

    pltpu.sync_copy(cls_hbm, cls_v)
    lo = cls_v[pl.ds(0, 16)][0]
    hi = cls_v[pl.ds(C - 16, 16)][15]

    def in_slice(g):
        return x_hbm.at[pl.ds(base + g * CHW, CHW)]

    def out_slice(g):
        return o_hbm.at[pl.ds(base + g * CHW, CHW)]

    lane_off = lax.iota(jnp.int32, 16) * D + 5

    def scan_fix(buf):
        @pl.loop(0, CH // 16)
        def _(sb):
            goff = sb * (16 * D)
            vals = plsc.load_gather(buf, [lane_off + goff])
            t = vals.astype(jnp.int32).astype(jnp.float32)
            ok = (vals == t) & (vals >= lo) & (vals <= hi)

            @pl.when(jnp.logical_not(jnp.all(ok)))
            def _():
                @pl.loop(0, 16)
                def _(k):
                    roff = goff + k * D
                    v = buf[pl.ds(roff, 16)][5]
                    vt = v.astype(jnp.int32).astype(jnp.float32)
                    good = (v == vt) & (v >= lo) & (v <= hi)

                    @pl.when(jnp.logical_not(good))
                    def _():
                        for j in range(D // 16):
                            buf[pl.ds(roff + 16 * j, 16)] = jnp.zeros(
                                (16,), jnp.float32)

    for p in range(PD):
        pltpu.async_copy(in_slice(p), bufs[p], sins[p])

    @pl.loop(0, CHUNKS, step=NBUF)
    def _(g0):
        for p in range(NBUF):
            g = g0 + p
            buf, si, so = bufs[p], sins[p], souts[p]
            q = (p + PD) % NBUF

            @pl.when(g + PD < CHUNKS)
            def _():
                @pl.when(g + PD - NBUF >= 0)
                def _():
                    pg = g + PD - NBUF
                    pltpu.make_async_copy(
                        bufs[q], out_slice(pg), souts[q]).wait()

                pltpu.async_copy(in_slice(g + PD), bufs[q], sins[q])

            pltpu.make_async_copy(in_slice(g), buf, si).wait()
            scan_fix(buf)
            pltpu.async_copy(buf, out_slice(g), so)

    for t in range(NBUF):
        g = CHUNKS - NBUF + t
        pltpu.make_async_copy(bufs[g % NBUF], out_slice(g),
                              souts[g % NBUF]).wait()


def kernel(x, classes):
    x1d = x.reshape(N * D)
    mesh = plsc.VectorSubcoreMesh(core_axis_name="c", subcore_axis_name="s")
    sc = pl.kernel(
        _sc_body,
        out_type=jax.ShapeDtypeStruct((N * D,), jnp.float32),
        mesh=mesh,
        compiler_params=pltpu.CompilerParams(needs_layout_passes=False),
        scratch_types=[
            pltpu.VMEM((CHW,), jnp.float32),
            pltpu.VMEM((CHW,), jnp.float32),
            pltpu.VMEM((CHW,), jnp.float32),
            pltpu.VMEM((CHW,), jnp.float32),
            pltpu.VMEM((C,), jnp.float32),
            pltpu.SemaphoreType.DMA,
            pltpu.SemaphoreType.DMA,
            pltpu.SemaphoreType.DMA,
            pltpu.SemaphoreType.DMA,
            pltpu.SemaphoreType.DMA,
            pltpu.SemaphoreType.DMA,
            pltpu.SemaphoreType.DMA,
            pltpu.SemaphoreType.DMA,
        ],
    )
    return sc(x1d, classes).reshape(N, D)
